# Initial kernel scaffold; baseline (speedup 1.0000x reference)
import jax
import jax.numpy as jnp
from jax.experimental import pallas as pl


def kernel(x):
    raise NotImplementedError("write your pallas kernel here")


# speedup vs baseline: 19.7489x; 19.7489x over previous
import functools

import jax
import jax.numpy as jnp
from jax.experimental import pallas as pl

B, N, D = 4, 2048, 256
TOPK = 16
R = 256
NBLK = N // R


def _normalize(v):
    nrm = jnp.sqrt(jnp.sum(v * v, axis=-1, keepdims=True))
    return v / jnp.maximum(nrm, 1e-12)


def _thresh_kernel(xb_ref, xf_ref, t_ref):
    nb = _normalize(xb_ref[0])
    nf = _normalize(xf_ref[0])
    s = jax.lax.dot_general(nb, nf, (((1,), (1,)), ((), ())),
                            preferred_element_type=jnp.float32)
    for _ in range(TOPK - 1):
        m = jnp.max(s, axis=1, keepdims=True)
        s = jnp.where(s >= m, -jnp.inf, s)
    t_ref[0, 0, :] = jnp.max(s, axis=1)


def _adj_kernel(xb_ref, xf_ref, tf_ref, o_ref):
    i = pl.program_id(1)
    nb = _normalize(xb_ref[0])
    nf = _normalize(xf_ref[0])
    s = jax.lax.dot_general(nb, nf, (((1,), (1,)), ((), ())),
                            preferred_element_type=jnp.float32)
    t_col = tf_ref[0, 0, :]
    t_row = tf_ref[0, 0, pl.ds(i * R, R)]
    tmin = jnp.minimum(t_row[:, None], t_col[None, :])
    adj = jnp.where(s >= tmin, 1.0, 0.0)
    rows = jax.lax.broadcasted_iota(jnp.int32, (R, N), 0) + i * R
    cols = jax.lax.broadcasted_iota(jnp.int32, (R, N), 1)
    o_ref[0] = jnp.where(rows == cols, 1.0, adj)


@jax.jit
def kernel(x):
    thresh = pl.pallas_call(
        _thresh_kernel,
        grid=(B, NBLK),
        in_specs=[
            pl.BlockSpec((1, R, D), lambda b, i: (b, i, 0)),
            pl.BlockSpec((1, N, D), lambda b, i: (b, 0, 0)),
        ],
        out_specs=pl.BlockSpec((1, 1, R), lambda b, i: (b * NBLK + i, 0, 0)),
        out_shape=jax.ShapeDtypeStruct((B * NBLK, 1, R), jnp.float32),
    )(x, x)
    thresh = thresh.reshape(B, 1, N)
    adj = pl.pallas_call(
        _adj_kernel,
        grid=(B, NBLK),
        in_specs=[
            pl.BlockSpec((1, R, D), lambda b, i: (b, i, 0)),
            pl.BlockSpec((1, N, D), lambda b, i: (b, 0, 0)),
            pl.BlockSpec((1, 1, N), lambda b, i: (b, 0, 0)),
        ],
        out_specs=pl.BlockSpec((1, R, N), lambda b, i: (b, i, 0)),
        out_shape=jax.ShapeDtypeStruct((B, N, N), jnp.float32),
    )(x, x, thresh)
    return adj


# speedup vs baseline: 23.1922x; 1.1743x over previous
import jax
import jax.numpy as jnp
from jax.experimental import pallas as pl

B, N, D = 4, 2048, 256
TOPK = 16
R = 512
NBLK = N // R
NR = 512


def _norm_kernel(x_ref, o_ref):
    v = x_ref[...]
    nrm = jnp.sqrt(jnp.sum(v * v, axis=-1, keepdims=True))
    o_ref[...] = v / jnp.maximum(nrm, 1e-12)


def _thresh_kernel(nb_ref, nf_ref, t_ref):
    s = jax.lax.dot_general(nb_ref[0], nf_ref[0], (((1,), (1,)), ((), ())),
                            preferred_element_type=jnp.float32)
    m = jnp.max(s, axis=1, keepdims=True)
    for _ in range(TOPK - 1):
        m = jnp.max(jnp.where(s < m, s, -jnp.inf), axis=1, keepdims=True)
    t_ref[0, 0, :] = m[:, 0]


def _adj_kernel(nb_ref, nf_ref, tf_ref, o_ref):
    i = pl.program_id(1)
    s = jax.lax.dot_general(nb_ref[0], nf_ref[0], (((1,), (1,)), ((), ())),
                            preferred_element_type=jnp.float32)
    t_col = tf_ref[0, 0, :]
    t_row = tf_ref[0, 0, pl.ds(i * R, R)]
    tmin = jnp.minimum(t_row[:, None], t_col[None, :])
    adj = jnp.where(s >= tmin, 1.0, 0.0)
    rows = jax.lax.broadcasted_iota(jnp.int32, (R, N), 0) + i * R
    cols = jax.lax.broadcasted_iota(jnp.int32, (R, N), 1)
    o_ref[0] = jnp.where(rows == cols, 1.0, adj)


@jax.jit
def kernel(x):
    normed = pl.pallas_call(
        _norm_kernel,
        grid=(B * N // NR,),
        in_specs=[pl.BlockSpec((NR, D), lambda i: (i, 0))],
        out_specs=pl.BlockSpec((NR, D), lambda i: (i, 0)),
        out_shape=jax.ShapeDtypeStruct((B * N, D), jnp.float32),
    )(x.reshape(B * N, D)).reshape(B, N, D)
    thresh = pl.pallas_call(
        _thresh_kernel,
        grid=(B, NBLK),
        in_specs=[
            pl.BlockSpec((1, R, D), lambda b, i: (b, i, 0)),
            pl.BlockSpec((1, N, D), lambda b, i: (b, 0, 0)),
        ],
        out_specs=pl.BlockSpec((1, 1, R), lambda b, i: (b * NBLK + i, 0, 0)),
        out_shape=jax.ShapeDtypeStruct((B * NBLK, 1, R), jnp.float32),
    )(normed, normed)
    thresh = thresh.reshape(B, 1, N)
    adj = pl.pallas_call(
        _adj_kernel,
        grid=(B, NBLK),
        in_specs=[
            pl.BlockSpec((1, R, D), lambda b, i: (b, i, 0)),
            pl.BlockSpec((1, N, D), lambda b, i: (b, 0, 0)),
            pl.BlockSpec((1, 1, N), lambda b, i: (b, 0, 0)),
        ],
        out_specs=pl.BlockSpec((1, R, N), lambda b, i: (b, i, 0)),
        out_shape=jax.ShapeDtypeStruct((B, N, N), jnp.float32),
    )(normed, normed, thresh)
    return adj


# speedup vs baseline: 35.0328x; 1.5105x over previous
import jax
import jax.numpy as jnp
from jax.experimental import pallas as pl

B, N, D = 4, 2048, 256
TOPK = 16
R = 512
NBLK = N // R
NR = 512


def _norm_kernel(x_ref, o_ref):
    v = x_ref[...]
    nrm = jnp.sqrt(jnp.sum(v * v, axis=-1, keepdims=True))
    o_ref[...] = v / jnp.maximum(nrm, 1e-12)


NLANE = 128
NCHUNK = N // NLANE


def _thresh_kernel(nb_ref, nf_ref, t_ref):
    s = jax.lax.dot_general(nb_ref[0], nf_ref[0], (((1,), (1,)), ((), ())),
                            preferred_element_type=jnp.float32)
    neg = jnp.full((R, NLANE), -jnp.inf, jnp.float32)
    h1 = h2 = h3 = h4 = neg
    for ci in range(NCHUNK):
        v = s[:, ci * NLANE:(ci + 1) * NLANE]
        m1 = jnp.maximum(h1, v); v = jnp.minimum(h1, v)
        m2 = jnp.maximum(h2, v); v = jnp.minimum(h2, v)
        m3 = jnp.maximum(h3, v); v = jnp.minimum(h3, v)
        m4 = jnp.maximum(h4, v)
        h1, h2, h3, h4 = m1, m2, m3, m4
    c = jnp.concatenate([h1, h2, h3, h4], axis=1)
    m = jnp.max(c, axis=1, keepdims=True)
    for _ in range(TOPK - 1):
        m = jnp.max(jnp.where(c < m, c, -jnp.inf), axis=1, keepdims=True)
    cnt = jnp.sum(jnp.where(s >= m, 1.0, 0.0), axis=1, keepdims=True)
    t_ref[0, 0, :] = m[:, 0]

    @pl.when(jnp.any(cnt > TOPK + 0.5))
    def _fallback():
        mf = jnp.max(s, axis=1, keepdims=True)
        for _ in range(TOPK - 1):
            mf = jnp.max(jnp.where(s < mf, s, -jnp.inf), axis=1, keepdims=True)
        t_ref[0, 0, :] = mf[:, 0]


def _adj_kernel(nb_ref, nf_ref, tf_ref, o_ref):
    i = pl.program_id(1)
    s = jax.lax.dot_general(nb_ref[0], nf_ref[0], (((1,), (1,)), ((), ())),
                            preferred_element_type=jnp.float32)
    t_col = tf_ref[0, 0, :]
    t_row = tf_ref[0, 0, pl.ds(i * R, R)]
    tmin = jnp.minimum(t_row[:, None], t_col[None, :])
    adj = jnp.where(s >= tmin, 1.0, 0.0)
    rows = jax.lax.broadcasted_iota(jnp.int32, (R, N), 0) + i * R
    cols = jax.lax.broadcasted_iota(jnp.int32, (R, N), 1)
    o_ref[0] = jnp.where(rows == cols, 1.0, adj)


@jax.jit
def kernel(x):
    normed = pl.pallas_call(
        _norm_kernel,
        grid=(B * N // NR,),
        in_specs=[pl.BlockSpec((NR, D), lambda i: (i, 0))],
        out_specs=pl.BlockSpec((NR, D), lambda i: (i, 0)),
        out_shape=jax.ShapeDtypeStruct((B * N, D), jnp.float32),
    )(x.reshape(B * N, D)).reshape(B, N, D)
    thresh = pl.pallas_call(
        _thresh_kernel,
        grid=(B, NBLK),
        in_specs=[
            pl.BlockSpec((1, R, D), lambda b, i: (b, i, 0)),
            pl.BlockSpec((1, N, D), lambda b, i: (b, 0, 0)),
        ],
        out_specs=pl.BlockSpec((1, 1, R), lambda b, i: (b * NBLK + i, 0, 0)),
        out_shape=jax.ShapeDtypeStruct((B * NBLK, 1, R), jnp.float32),
    )(normed, normed)
    thresh = thresh.reshape(B, 1, N)
    adj = pl.pallas_call(
        _adj_kernel,
        grid=(B, NBLK),
        in_specs=[
            pl.BlockSpec((1, R, D), lambda b, i: (b, i, 0)),
            pl.BlockSpec((1, N, D), lambda b, i: (b, 0, 0)),
            pl.BlockSpec((1, 1, N), lambda b, i: (b, 0, 0)),
        ],
        out_specs=pl.BlockSpec((1, R, N), lambda b, i: (b, i, 0)),
        out_shape=jax.ShapeDtypeStruct((B, N, N), jnp.float32),
    )(normed, normed, thresh)
    return adj


# speedup vs baseline: 40.1140x; 1.1450x over previous
import jax
import jax.numpy as jnp
from jax.experimental import pallas as pl
from jax.experimental.pallas import tpu as pltpu

B, N, D = 4, 2048, 256
TOPK = 16
R = 512
NBLK = N // R
NLANE = 128
NCHUNK = N // NLANE


def _fused_kernel(x_ref, o_ref, n_scr, t_scr, s_scr):
    p = pl.program_id(1)
    i = pl.program_id(2)

    @pl.when((p == 0) & (i == 0))
    def _normalize():
        v = x_ref[0]
        nrm = jnp.sqrt(jnp.sum(v * v, axis=-1, keepdims=True))
        n_scr[...] = v / jnp.maximum(nrm, 1e-12)

    @pl.when(p == 0)
    def _thresh():
        nb = n_scr[pl.ds(i * R, R), :]
        s = jax.lax.dot_general(nb, n_scr[...], (((1,), (1,)), ((), ())),
                                preferred_element_type=jnp.float32)
        s_scr[pl.ds(i * R, R), :] = s
        neg = jnp.full((R, NLANE), -jnp.inf, jnp.float32)
        h1 = h2 = h3 = h4 = neg
        for ci in range(NCHUNK):
            v = s[:, ci * NLANE:(ci + 1) * NLANE]
            m1 = jnp.maximum(h1, v); v = jnp.minimum(h1, v)
            m2 = jnp.maximum(h2, v); v = jnp.minimum(h2, v)
            m3 = jnp.maximum(h3, v); v = jnp.minimum(h3, v)
            m4 = jnp.maximum(h4, v)
            h1, h2, h3, h4 = m1, m2, m3, m4
        c = jnp.concatenate([h1, h2, h3, h4], axis=1)
        m = jnp.max(h1, axis=1, keepdims=True)
        for _ in range(TOPK - 1):
            m = jnp.max(jnp.where(c < m, c, -jnp.inf), axis=1, keepdims=True)
        cnt = jnp.sum(jnp.where(s >= m, 1.0, 0.0), axis=1, keepdims=True)
        t_scr[0, pl.ds(i * R, R)] = m[:, 0]

        @pl.when(jnp.any(cnt > TOPK + 0.5))
        def _fallback():
            mf = jnp.max(s, axis=1, keepdims=True)
            for _ in range(TOPK - 1):
                mf = jnp.max(jnp.where(s < mf, s, -jnp.inf),
                             axis=1, keepdims=True)
            t_scr[0, pl.ds(i * R, R)] = mf[:, 0]

    @pl.when(p == 1)
    def _adj():
        s = s_scr[pl.ds(i * R, R), :]
        t_col = t_scr[0, :]
        t_row = t_scr[0, pl.ds(i * R, R)]
        tmin = jnp.minimum(t_row[:, None], t_col[None, :])
        adj = jnp.where(s >= tmin, 1.0, 0.0)
        rows = jax.lax.broadcasted_iota(jnp.int32, (R, N), 0) + i * R
        cols = jax.lax.broadcasted_iota(jnp.int32, (R, N), 1)
        o_ref[0] = jnp.where(rows == cols, 1.0, adj)


@jax.jit
def kernel(x):
    return pl.pallas_call(
        _fused_kernel,
        grid=(B, 2, NBLK),
        in_specs=[pl.BlockSpec((1, N, D), lambda b, p, i: (b, 0, 0))],
        out_specs=pl.BlockSpec((1, R, N), lambda b, p, i: (b, i * p, 0)),
        out_shape=jax.ShapeDtypeStruct((B, N, N), jnp.float32),
        scratch_shapes=[
            pltpu.VMEM((N, D), jnp.float32),
            pltpu.VMEM((1, N), jnp.float32),
            pltpu.VMEM((N, N), jnp.float32),
        ],
    )(x)
